# baseline (speedup 1.0000x reference)
import jax
import jax.numpy as jnp
from jax import lax
from jax.experimental import pallas as pl
from jax.experimental.pallas import tpu as pltpu
from jax.experimental.pallas import tpu_sc as plsc

_HID = 32
_EBLK = 128


def _pre_body(x_ref, xh_ref, h_ref, w0_ref, w1_ref, w2_ref, b_ref, o_ref):
    acc = x_ref[...] * w0_ref[...]
    acc = acc + jnp.dot(xh_ref[...], w1_ref[...], preferred_element_type=jnp.float32)
    acc = acc + jnp.dot(h_ref[...], w2_ref[...], preferred_element_type=jnp.float32)
    o_ref[...] = acc + b_ref[...]


def _pre_tc(x, xh, h2, W_in, b_in):
    n = x.shape[0]
    B = 1000
    return pl.pallas_call(
        _pre_body,
        grid=(n // B,),
        in_specs=[
            pl.BlockSpec((B, 1), lambda i: (i, 0)),
            pl.BlockSpec((B, 96), lambda i: (i, 0)),
            pl.BlockSpec((B, _HID), lambda i: (i, 0)),
            pl.BlockSpec((1, _HID), lambda i: (0, 0)),
            pl.BlockSpec((96, _HID), lambda i: (0, 0)),
            pl.BlockSpec((_HID, _HID), lambda i: (0, 0)),
            pl.BlockSpec((1, _HID), lambda i: (0, 0)),
        ],
        out_specs=pl.BlockSpec((B, _HID), lambda i: (i, 0)),
        out_shape=jax.ShapeDtypeStruct((n, _HID), jnp.float32),
    )(x, xh, h2, W_in[0:1], W_in[1:97], W_in[97:129], b_in.reshape(1, _HID))


def _edge_pass(xin, gat, red, w, n_nodes):
    E = w.shape[0]
    H = n_nodes // 2
    HP = ((H + 16 + 1023) // 1024) * 1024
    TROWS = HP // 16
    nblk = E // _EBLK

    EB3 = 3 * _EBLK * 4
    GB = _EBLK * _HID * 4
    SCB = GB + _EBLK * 4

    def body(xin_r, gat_r, red_r, w_r, acc_o, deg_o,
             gidx_a, sidx_a, w_a, rows_a, gidx_b, sidx_b, w_b, rows_b,
             zrows_v, zdeg_v, acc_sh, deg_sh,
             sem_ea, sem_eb, sem_ga, sem_gb, sem_sa, sem_sb):
        c = lax.axis_index("c")
        s = lax.axis_index("s")
        base = c * H
        row0 = s * TROWS
        zero16 = jnp.zeros((16,), jnp.float32)
        for i in range(16):
            zrows_v[i, pl.ds(0, 16)] = zero16
            zrows_v[i, pl.ds(16, 16)] = zero16
        for i in range(4):
            zdeg_v[pl.ds(16 * i, 16)] = zero16

        def zrow_body(r, carry):
            pltpu.sync_copy(zrows_v, acc_sh.at[pl.ds(row0 + r * 16, 16)])
            return carry

        lax.fori_loop(0, TROWS // 16, zrow_body, 0)

        def zdeg_body(r, carry):
            pltpu.sync_copy(zdeg_v, deg_sh.at[pl.ds(row0 + r * 64, 64)])
            return carry

        lax.fori_loop(0, TROWS // 64, zdeg_body, 0)
        plsc.subcore_barrier()

        iota = lax.iota(jnp.int32, 16)
        nb = (nblk - s + 15) // 16

        def _prep(gidx_v, sidx_v, w_v):
            for k in range(_EBLK // 16):
                g16 = gidx_v[pl.ds(16 * k, 16)]
                t16 = sidx_v[pl.ds(16 * k, 16)]
                w16 = w_v[pl.ds(16 * k, 16)]
                w16 = jnp.where(g16 != t16, w16, jnp.float32(0.0))
                inr = (t16 >= base) & (t16 < base + H)
                sidx_v[pl.ds(16 * k, 16)] = jnp.where(inr, t16 - base, H + iota)
                w_v[pl.ds(16 * k, 16)] = w16

        splats = [jnp.full((16, 1), l, jnp.int32) for l in range(16)]
        gdims = lax.GatherDimensionNumbers(
            offset_dims=(), collapsed_slice_dims=(0,), start_index_map=(0,))

        def _scale(rows_v, w_v):
            for k in range(_EBLK // 16):
                w16 = w_v[pl.ds(16 * k, 16)]
                for l in range(16):
                    wb = lax.gather(w16, splats[l], gdims, (1,),
                                    mode=lax.GatherScatterMode.PROMISE_IN_BOUNDS)
                    r = 16 * k + l
                    rows_v[r, pl.ds(0, 16)] = rows_v[r, pl.ds(0, 16)] * wb
                    rows_v[r, pl.ds(16, 16)] = rows_v[r, pl.ds(16, 16)] * wb

        def _edges(i, gidx_v, sidx_v, w_v, sem_e):
            e0 = pl.multiple_of((s + i * 16) * _EBLK, _EBLK)
            cps = [pltpu.async_copy(gat_r.at[pl.ds(e0, _EBLK)], gidx_v, sem_e),
                   pltpu.async_copy(red_r.at[pl.ds(e0, _EBLK)], sidx_v, sem_e),
                   pltpu.async_copy(w_r.at[pl.ds(e0, _EBLK)], w_v, sem_e)]
            for cp in cps:
                cp.wait()

        ng = (nb + 1) // 2

        def grp_body(g, carry):
            i0 = 2 * g
            i1 = 2 * g + 1

            @pl.when(g > 0)
            def _():
                pltpu.make_async_copy(rows_a, acc_sh.at[sidx_a], sem_sa).wait()
                pltpu.make_async_copy(w_a, deg_sh.at[sidx_a], sem_sa).wait()

            _edges(i0, gidx_a, sidx_a, w_a, sem_ea)
            _prep(gidx_a, sidx_a, w_a)
            pltpu.async_copy(xin_r.at[gidx_a], rows_a, sem_ga)

            @pl.when(i1 < nb)
            def _():
                @pl.when(g > 0)
                def _():
                    pltpu.make_async_copy(rows_b, acc_sh.at[sidx_b], sem_sb).wait()
                    pltpu.make_async_copy(w_b, deg_sh.at[sidx_b], sem_sb).wait()

                _edges(i1, gidx_b, sidx_b, w_b, sem_eb)
                _prep(gidx_b, sidx_b, w_b)
                pltpu.async_copy(xin_r.at[gidx_b], rows_b, sem_gb)

            pltpu.make_async_copy(xin_r.at[gidx_a], rows_a, sem_ga).wait()
            _scale(rows_a, w_a)
            pltpu.async_copy(rows_a, acc_sh.at[sidx_a], sem_sa, add=True)
            pltpu.async_copy(w_a, deg_sh.at[sidx_a], sem_sa, add=True)

            @pl.when(i1 < nb)
            def _():
                pltpu.make_async_copy(xin_r.at[gidx_b], rows_b, sem_gb).wait()
                _scale(rows_b, w_b)
                pltpu.async_copy(rows_b, acc_sh.at[sidx_b], sem_sb, add=True)
                pltpu.async_copy(w_b, deg_sh.at[sidx_b], sem_sb, add=True)
            return carry

        lax.fori_loop(0, ng, grp_body, 0)
        pltpu.make_async_copy(rows_a, acc_sh.at[sidx_a], sem_sa).wait()
        pltpu.make_async_copy(w_a, deg_sh.at[sidx_a], sem_sa).wait()
        pltpu.make_async_copy(rows_b, acc_sh.at[sidx_b], sem_sb).wait()
        pltpu.make_async_copy(w_b, deg_sh.at[sidx_b], sem_sb).wait()
        plsc.subcore_barrier()
        out0 = pl.multiple_of(c * HP + row0, 8)
        pltpu.sync_copy(acc_sh.at[pl.ds(row0, TROWS)], acc_o.at[pl.ds(out0, TROWS)])
        pltpu.sync_copy(deg_sh.at[pl.ds(row0, TROWS)], deg_o.at[pl.ds(out0, TROWS)])

    mesh = plsc.VectorSubcoreMesh(core_axis_name="c", subcore_axis_name="s")
    acc, deg = pl.kernel(
        body,
        mesh=mesh,
        compiler_params=pltpu.CompilerParams(use_tc_tiling_on_sc=False),
        out_type=[
            jax.ShapeDtypeStruct((2 * HP, _HID), jnp.float32),
            jax.ShapeDtypeStruct((2 * HP,), jnp.float32),
        ],
        scratch_types=[
            pltpu.VMEM((_EBLK,), jnp.int32),
            pltpu.VMEM((_EBLK,), jnp.int32),
            pltpu.VMEM((_EBLK,), jnp.float32),
            pltpu.VMEM((_EBLK, _HID), jnp.float32),
            pltpu.VMEM((_EBLK,), jnp.int32),
            pltpu.VMEM((_EBLK,), jnp.int32),
            pltpu.VMEM((_EBLK,), jnp.float32),
            pltpu.VMEM((_EBLK, _HID), jnp.float32),
            pltpu.VMEM((16, _HID), jnp.float32),
            pltpu.VMEM((64,), jnp.float32),
            pltpu.VMEM_SHARED((HP, _HID), jnp.float32),
            pltpu.VMEM_SHARED((HP,), jnp.float32),
            pltpu.SemaphoreType.DMA,
            pltpu.SemaphoreType.DMA,
            pltpu.SemaphoreType.DMA,
            pltpu.SemaphoreType.DMA,
            pltpu.SemaphoreType.DMA,
            pltpu.SemaphoreType.DMA,
        ],
    )(xin, gat, red, w)
    acc = acc.reshape(2, HP, _HID)[:, :H].reshape(n_nodes, _HID)
    deg = deg.reshape(2, HP)[:, :H].reshape(n_nodes)
    return acc, deg


def _post_body(af_ref, ab_ref, di_ref, do_ref, h_ref,
               wf1_ref, wf2_ref, bf_ref, wmu_ref, wsg_ref, bsg_ref,
               wpi_ref, bpi_ref, gmm_ref, o1_ref):
    di = di_ref[...]
    do = do_ref[...]
    of = af_ref[...] * (1.0 / jnp.where(di == 0.0, 1.0, di))
    ob = ab_ref[...] * (1.0 / jnp.where(do == 0.0, 1.0, do))
    out = (jnp.dot(of, wf1_ref[...], preferred_element_type=jnp.float32)
           + jnp.dot(ob, wf2_ref[...], preferred_element_type=jnp.float32)
           + bf_ref[...])
    o1 = jnp.concatenate([out, h_ref[...]], axis=-1)
    o1_ref[...] = o1
    mu = jnp.dot(o1, wmu_ref[...], preferred_element_type=jnp.float32)
    zs = jnp.dot(o1, wsg_ref[...], preferred_element_type=jnp.float32) + bsg_ref[...]
    sigma = jnp.maximum(zs, 0.0) + jnp.log1p(jnp.exp(-jnp.abs(zs)))
    zp = jnp.dot(o1, wpi_ref[...], preferred_element_type=jnp.float32) + bpi_ref[...]
    zp = zp - jnp.max(zp, axis=-1, keepdims=True)
    ez = jnp.exp(zp)
    pi = ez / jnp.sum(ez, axis=-1, keepdims=True)
    gmm_ref[...] = jnp.concatenate([mu, sigma, pi], axis=-1)


def _post_tc(af, ab, di, do, h2, wf1, wf2, bf, wmu, wsg, bsg, wpi, bpi):
    n = af.shape[0]
    B = 1000
    full = lambda r, c: pl.BlockSpec((r, c), lambda i: (0, 0))
    blk = lambda c: pl.BlockSpec((B, c), lambda i: (i, 0))
    return pl.pallas_call(
        _post_body,
        grid=(n // B,),
        in_specs=[
            blk(_HID), blk(_HID), blk(1), blk(1), blk(_HID),
            full(_HID, _HID), full(_HID, _HID), full(1, _HID),
            full(2 * _HID, _HID), full(2 * _HID, _HID), full(1, _HID),
            full(2 * _HID, _HID), full(1, _HID),
        ],
        out_specs=[blk(3 * _HID), blk(2 * _HID)],
        out_shape=[
            jax.ShapeDtypeStruct((n, 3 * _HID), jnp.float32),
            jax.ShapeDtypeStruct((n, 2 * _HID), jnp.float32),
        ],
    )(af, ab, di, do, h2, wf1, wf2, bf, wmu, wsg, bsg, wpi, bpi)


def kernel(x, x_hat_1, h, edge_index, edge_weight,
           W_in, b_in, W_filt, b_filt, W_mu, W_sigma, b_sigma, W_pi, b_pi):
    n = x.shape[0]
    h2 = h.reshape(n, _HID)
    xin = _pre_tc(x, x_hat_1, h2, W_in, b_in)
    src = edge_index[0]
    dst = edge_index[1]
    accf, degi = _edge_pass(xin, src, dst, edge_weight, n)
    accb, dego = _edge_pass(xin, dst, src, edge_weight, n)
    gmm, out1 = _post_tc(
        accf, accb, degi.reshape(n, 1), dego.reshape(n, 1), h2,
        W_filt[:_HID], W_filt[_HID:], b_filt.reshape(1, _HID),
        W_mu, W_sigma, b_sigma.reshape(1, _HID), W_pi, b_pi.reshape(1, _HID))
    return gmm, out1.reshape(n, 1, 2 * _HID), h


# speedup vs baseline: 1.0427x; 1.0427x over previous
import jax
import jax.numpy as jnp
from jax import lax
from jax.experimental import pallas as pl
from jax.experimental.pallas import tpu as pltpu
from jax.experimental.pallas import tpu_sc as plsc

_HID = 32
_EBLK = 128
_BPOST = 400


def _pre_body(x_ref, xh_ref, h_ref, w0_ref, w1_ref, w2_ref, b_ref, o_ref):
    acc = x_ref[...] * w0_ref[...]
    acc = acc + jnp.dot(xh_ref[...], w1_ref[...], preferred_element_type=jnp.float32)
    acc = acc + jnp.dot(h_ref[...], w2_ref[...], preferred_element_type=jnp.float32)
    o_ref[...] = acc + b_ref[...]


def _pre_tc(x, xh, h2, W_in, b_in):
    n = x.shape[0]
    B = 1000
    return pl.pallas_call(
        _pre_body,
        grid=(n // B,),
        in_specs=[
            pl.BlockSpec((B, 1), lambda i: (i, 0)),
            pl.BlockSpec((B, 96), lambda i: (i, 0)),
            pl.BlockSpec((B, _HID), lambda i: (i, 0)),
            pl.BlockSpec((1, _HID), lambda i: (0, 0)),
            pl.BlockSpec((96, _HID), lambda i: (0, 0)),
            pl.BlockSpec((_HID, _HID), lambda i: (0, 0)),
            pl.BlockSpec((1, _HID), lambda i: (0, 0)),
        ],
        out_specs=pl.BlockSpec((B, _HID), lambda i: (i, 0)),
        out_shape=jax.ShapeDtypeStruct((n, _HID), jnp.float32),
    )(x, xh, h2, W_in[0:1], W_in[1:97], W_in[97:129], b_in.reshape(1, _HID))


def _edge_kernel(xin, src, dst, w, n_nodes):
    E = w.shape[0]
    H = n_nodes // 2
    HP = ((H + 16 + 1599) // 1600) * 1600
    TROWS = HP // 16
    nblk = E // _EBLK

    def body(xin_r, src_r, dst_r, w_r, accf_o, degi_o, accb_o, dego_o,
             gidx_a, sidx_a, w_a, rows_a, gidx_b, sidx_b, w_b, rows_b,
             zrows_v, zdeg_v, acc_sh, deg_sh,
             sem_ea, sem_eb, sem_ga, sem_gb, sem_sa, sem_sb):
        c = lax.axis_index("c")
        s = lax.axis_index("s")
        base = c * H
        row0 = s * TROWS
        iota = lax.iota(jnp.int32, 16)
        nb = (nblk - s + 15) // 16
        ng = (nb + 1) // 2
        zero16 = jnp.zeros((16,), jnp.float32)
        for i in range(16):
            zrows_v[i, pl.ds(0, 16)] = zero16
            zrows_v[i, pl.ds(16, 16)] = zero16
        for i in range(4):
            zdeg_v[pl.ds(16 * i, 16)] = zero16

        splats = [jnp.full((16, 1), l, jnp.int32) for l in range(16)]
        gdims = lax.GatherDimensionNumbers(
            offset_dims=(), collapsed_slice_dims=(0,), start_index_map=(0,))

        def _prep(gidx_v, sidx_v, w_v):
            for k in range(_EBLK // 16):
                g16 = gidx_v[pl.ds(16 * k, 16)]
                t16 = sidx_v[pl.ds(16 * k, 16)]
                w16 = w_v[pl.ds(16 * k, 16)]
                w16 = jnp.where(g16 != t16, w16, jnp.float32(0.0))
                inr = (t16 >= base) & (t16 < base + H)
                sidx_v[pl.ds(16 * k, 16)] = jnp.where(inr, t16 - base, H + iota)
                w_v[pl.ds(16 * k, 16)] = w16

        def _scale(rows_v, w_v):
            def kbody(k, carry):
                off = pl.multiple_of(16 * k, 16)
                w16 = w_v[pl.ds(off, 16)]
                for l in range(16):
                    wb = lax.gather(w16, splats[l], gdims, (1,),
                                    mode=lax.GatherScatterMode.PROMISE_IN_BOUNDS)
                    r = 16 * k + l
                    rows_v[r, pl.ds(0, 16)] = rows_v[r, pl.ds(0, 16)] * wb
                    rows_v[r, pl.ds(16, 16)] = rows_v[r, pl.ds(16, 16)] * wb
                return carry

            lax.fori_loop(0, _EBLK // 16, kbody, 0)

        def _phase(gat_r, red_r, acc_o, deg_o):
            def zrow_body(r, carry):
                pltpu.sync_copy(zrows_v, acc_sh.at[pl.ds(row0 + r * 16, 16)])
                return carry

            lax.fori_loop(0, TROWS // 16, zrow_body, 0)

            def zdeg_body(r, carry):
                pltpu.sync_copy(zdeg_v, deg_sh.at[pl.ds(row0 + r * 64, 64)])
                return carry

            lax.fori_loop(0, TROWS // 64, zdeg_body, 0)
            plsc.subcore_barrier()

            def _edges(i, gidx_v, sidx_v, w_v, sem_e):
                e0 = pl.multiple_of((s + i * 16) * _EBLK, _EBLK)
                cps = [pltpu.async_copy(gat_r.at[pl.ds(e0, _EBLK)], gidx_v, sem_e),
                       pltpu.async_copy(red_r.at[pl.ds(e0, _EBLK)], sidx_v, sem_e),
                       pltpu.async_copy(w_r.at[pl.ds(e0, _EBLK)], w_v, sem_e)]
                for cp in cps:
                    cp.wait()

            def grp_body(g, carry):
                i0 = 2 * g
                i1 = 2 * g + 1

                @pl.when(g > 0)
                def _():
                    pltpu.make_async_copy(rows_a, acc_sh.at[sidx_a], sem_sa).wait()
                    pltpu.make_async_copy(w_a, deg_sh.at[sidx_a], sem_sa).wait()

                _edges(i0, gidx_a, sidx_a, w_a, sem_ea)
                _prep(gidx_a, sidx_a, w_a)
                pltpu.async_copy(xin_r.at[gidx_a], rows_a, sem_ga)

                @pl.when(i1 < nb)
                def _():
                    @pl.when(g > 0)
                    def _():
                        pltpu.make_async_copy(rows_b, acc_sh.at[sidx_b], sem_sb).wait()
                        pltpu.make_async_copy(w_b, deg_sh.at[sidx_b], sem_sb).wait()

                    _edges(i1, gidx_b, sidx_b, w_b, sem_eb)
                    _prep(gidx_b, sidx_b, w_b)
                    pltpu.async_copy(xin_r.at[gidx_b], rows_b, sem_gb)

                pltpu.make_async_copy(xin_r.at[gidx_a], rows_a, sem_ga).wait()
                _scale(rows_a, w_a)
                pltpu.async_copy(rows_a, acc_sh.at[sidx_a], sem_sa, add=True)
                pltpu.async_copy(w_a, deg_sh.at[sidx_a], sem_sa, add=True)

                @pl.when(i1 < nb)
                def _():
                    pltpu.make_async_copy(xin_r.at[gidx_b], rows_b, sem_gb).wait()
                    _scale(rows_b, w_b)
                    pltpu.async_copy(rows_b, acc_sh.at[sidx_b], sem_sb, add=True)
                    pltpu.async_copy(w_b, deg_sh.at[sidx_b], sem_sb, add=True)
                return carry

            lax.fori_loop(0, ng, grp_body, 0)
            pltpu.make_async_copy(rows_a, acc_sh.at[sidx_a], sem_sa).wait()
            pltpu.make_async_copy(w_a, deg_sh.at[sidx_a], sem_sa).wait()
            pltpu.make_async_copy(rows_b, acc_sh.at[sidx_b], sem_sb).wait()
            pltpu.make_async_copy(w_b, deg_sh.at[sidx_b], sem_sb).wait()
            plsc.subcore_barrier()
            out0 = pl.multiple_of(c * HP + row0, 8)
            pltpu.sync_copy(acc_sh.at[pl.ds(row0, TROWS)], acc_o.at[pl.ds(out0, TROWS)])
            pltpu.sync_copy(deg_sh.at[pl.ds(row0, TROWS)], deg_o.at[pl.ds(out0, TROWS)])

        _phase(src_r, dst_r, accf_o, degi_o)
        _phase(dst_r, src_r, accb_o, dego_o)

    mesh = plsc.VectorSubcoreMesh(core_axis_name="c", subcore_axis_name="s")
    return pl.kernel(
        body,
        mesh=mesh,
        compiler_params=pltpu.CompilerParams(use_tc_tiling_on_sc=False),
        out_type=[
            jax.ShapeDtypeStruct((2 * HP, _HID), jnp.float32),
            jax.ShapeDtypeStruct((2 * HP,), jnp.float32),
            jax.ShapeDtypeStruct((2 * HP, _HID), jnp.float32),
            jax.ShapeDtypeStruct((2 * HP,), jnp.float32),
        ],
        scratch_types=[
            pltpu.VMEM((_EBLK,), jnp.int32),
            pltpu.VMEM((_EBLK,), jnp.int32),
            pltpu.VMEM((_EBLK,), jnp.float32),
            pltpu.VMEM((_EBLK, _HID), jnp.float32),
            pltpu.VMEM((_EBLK,), jnp.int32),
            pltpu.VMEM((_EBLK,), jnp.int32),
            pltpu.VMEM((_EBLK,), jnp.float32),
            pltpu.VMEM((_EBLK, _HID), jnp.float32),
            pltpu.VMEM((16, _HID), jnp.float32),
            pltpu.VMEM((64,), jnp.float32),
            pltpu.VMEM_SHARED((HP, _HID), jnp.float32),
            pltpu.VMEM_SHARED((HP,), jnp.float32),
            pltpu.SemaphoreType.DMA,
            pltpu.SemaphoreType.DMA,
            pltpu.SemaphoreType.DMA,
            pltpu.SemaphoreType.DMA,
            pltpu.SemaphoreType.DMA,
            pltpu.SemaphoreType.DMA,
        ],
    )(xin, src, dst, w)


def _post_body(af_ref, ab_ref, di_ref, do_ref, h_ref,
               wf1_ref, wf2_ref, bf_ref, wmu_ref, wsg_ref, bsg_ref,
               wpi_ref, bpi_ref, gmm_ref, o1_ref):
    di = di_ref[...]
    do = do_ref[...]
    of = af_ref[...] * (1.0 / jnp.where(di == 0.0, 1.0, di))
    ob = ab_ref[...] * (1.0 / jnp.where(do == 0.0, 1.0, do))
    out = (jnp.dot(of, wf1_ref[...], preferred_element_type=jnp.float32)
           + jnp.dot(ob, wf2_ref[...], preferred_element_type=jnp.float32)
           + bf_ref[...])
    o1 = jnp.concatenate([out, h_ref[...]], axis=-1)
    o1_ref[...] = o1
    mu = jnp.dot(o1, wmu_ref[...], preferred_element_type=jnp.float32)
    zs = jnp.dot(o1, wsg_ref[...], preferred_element_type=jnp.float32) + bsg_ref[...]
    sigma = jnp.maximum(zs, 0.0) + jnp.log1p(jnp.exp(-jnp.abs(zs)))
    zp = jnp.dot(o1, wpi_ref[...], preferred_element_type=jnp.float32) + bpi_ref[...]
    zp = zp - jnp.max(zp, axis=-1, keepdims=True)
    ez = jnp.exp(zp)
    pi = ez / jnp.sum(ez, axis=-1, keepdims=True)
    gmm_ref[...] = jnp.concatenate([mu, sigma, pi], axis=-1)


def _post_tc(af, ab, di, do, h2, wf1, wf2, bf, wmu, wsg, bsg, wpi, bpi, n):
    HP = af.shape[0] // 2
    H = n // 2
    realb = H // _BPOST
    padb = (HP - H) // _BPOST
    pad_map = lambda i: (i + padb * (i >= realb).astype(jnp.int32), 0)
    full = lambda r, c: pl.BlockSpec((r, c), lambda i: (0, 0))
    blk = lambda c: pl.BlockSpec((_BPOST, c), lambda i: (i, 0))
    pblk = lambda c: pl.BlockSpec((_BPOST, c), pad_map)
    return pl.pallas_call(
        _post_body,
        grid=(n // _BPOST,),
        in_specs=[
            pblk(_HID), pblk(_HID), pblk(1), pblk(1), blk(_HID),
            full(_HID, _HID), full(_HID, _HID), full(1, _HID),
            full(2 * _HID, _HID), full(2 * _HID, _HID), full(1, _HID),
            full(2 * _HID, _HID), full(1, _HID),
        ],
        out_specs=[blk(3 * _HID), blk(2 * _HID)],
        out_shape=[
            jax.ShapeDtypeStruct((n, 3 * _HID), jnp.float32),
            jax.ShapeDtypeStruct((n, 2 * _HID), jnp.float32),
        ],
    )(af, ab, di, do, h2, wf1, wf2, bf, wmu, wsg, bsg, wpi, bpi)


def kernel(x, x_hat_1, h, edge_index, edge_weight,
           W_in, b_in, W_filt, b_filt, W_mu, W_sigma, b_sigma, W_pi, b_pi):
    n = x.shape[0]
    h2 = h.reshape(n, _HID)
    xin = _pre_tc(x, x_hat_1, h2, W_in, b_in)
    src = edge_index[0]
    dst = edge_index[1]
    accf, degi, accb, dego = _edge_kernel(xin, src, dst, edge_weight, n)
    gmm, out1 = _post_tc(
        accf, accb, degi.reshape(-1, 1), dego.reshape(-1, 1), h2,
        W_filt[:_HID], W_filt[_HID:], b_filt.reshape(1, _HID),
        W_mu, W_sigma, b_sigma.reshape(1, _HID), W_pi, b_pi.reshape(1, _HID), n)
    return gmm, out1.reshape(n, 1, 2 * _HID), h


# speedup vs baseline: 1.1451x; 1.0982x over previous
import jax
import jax.numpy as jnp
from jax import lax
from jax.experimental import pallas as pl
from jax.experimental.pallas import tpu as pltpu
from jax.experimental.pallas import tpu_sc as plsc

_HID = 32
_EBLK = 128
_BPOST = 400


def _pre_body(x_ref, xh_ref, h_ref, w0_ref, w1_ref, w2_ref, b_ref, o_ref):
    acc = x_ref[...] * w0_ref[...]
    acc = acc + jnp.dot(xh_ref[...], w1_ref[...], preferred_element_type=jnp.float32)
    acc = acc + jnp.dot(h_ref[...], w2_ref[...], preferred_element_type=jnp.float32)
    o_ref[...] = acc + b_ref[...]


def _pre_tc(x, xh, h2, W_in, b_in):
    n = x.shape[0]
    B = 1000
    return pl.pallas_call(
        _pre_body,
        grid=(n // B,),
        in_specs=[
            pl.BlockSpec((B, 1), lambda i: (i, 0)),
            pl.BlockSpec((B, 96), lambda i: (i, 0)),
            pl.BlockSpec((B, _HID), lambda i: (i, 0)),
            pl.BlockSpec((1, _HID), lambda i: (0, 0)),
            pl.BlockSpec((96, _HID), lambda i: (0, 0)),
            pl.BlockSpec((_HID, _HID), lambda i: (0, 0)),
            pl.BlockSpec((1, _HID), lambda i: (0, 0)),
        ],
        out_specs=pl.BlockSpec((B, _HID), lambda i: (i, 0)),
        out_shape=jax.ShapeDtypeStruct((n, _HID), jnp.float32),
    )(x, xh, h2, W_in[0:1], W_in[1:97], W_in[97:129], b_in.reshape(1, _HID))


def _edge_kernel(xin, src, dst, w, n_nodes):
    E = w.shape[0]
    H = n_nodes // 2
    HP = ((H + 16 + 1599) // 1600) * 1600
    TROWS = HP // 16
    nblk = E // _EBLK

    def body(xin_r, src_r, dst_r, w_r, accf_o, degi_o, accb_o, dego_o,
             gidx_a, sidx_a, w_a, rows_a, gidx_b, sidx_b, w_b, rows_b,
             zrows_v, zdeg_v, acc_sh, deg_sh,
             sem_ea, sem_eb, sem_ga, sem_gb, sem_sa, sem_sb):
        c = lax.axis_index("c")
        s = lax.axis_index("s")
        base = c * H
        row0 = s * TROWS
        iota = lax.iota(jnp.int32, 16)
        nb = (nblk - s + 15) // 16
        ng = (nb + 1) // 2
        zero16 = jnp.zeros((16,), jnp.float32)
        for i in range(16):
            zrows_v[i, pl.ds(0, 16)] = zero16
            zrows_v[i, pl.ds(16, 16)] = zero16
        for i in range(4):
            zdeg_v[pl.ds(16 * i, 16)] = zero16

        splats = [jnp.full((16, 1), l, jnp.int32) for l in range(16)]
        gdims = lax.GatherDimensionNumbers(
            offset_dims=(), collapsed_slice_dims=(0,), start_index_map=(0,))

        def _prep(gidx_v, sidx_v, w_v):
            for k in range(_EBLK // 16):
                g16 = gidx_v[pl.ds(16 * k, 16)]
                t16 = sidx_v[pl.ds(16 * k, 16)]
                w16 = w_v[pl.ds(16 * k, 16)]
                w16 = jnp.where(g16 != t16, w16, jnp.float32(0.0))
                inr = (t16 >= base) & (t16 < base + H)
                sidx_v[pl.ds(16 * k, 16)] = jnp.where(inr, t16 - base, H + iota)
                w_v[pl.ds(16 * k, 16)] = w16

        def _scale(rows_v, w_v):
            def kbody(k, carry):
                off = pl.multiple_of(16 * k, 16)
                w16 = w_v[pl.ds(off, 16)]
                for l in range(16):
                    wb = lax.gather(w16, splats[l], gdims, (1,),
                                    mode=lax.GatherScatterMode.PROMISE_IN_BOUNDS)
                    r = 16 * k + l
                    rows_v[r, pl.ds(0, 16)] = rows_v[r, pl.ds(0, 16)] * wb
                    rows_v[r, pl.ds(16, 16)] = rows_v[r, pl.ds(16, 16)] * wb
                return carry

            lax.fori_loop(0, _EBLK // 16, kbody, 0)

        def _phase(gat_r, red_r, acc_o, deg_o):
            def zrow_body(r, carry):
                pltpu.sync_copy(zrows_v, acc_sh.at[pl.ds(row0 + r * 16, 16)])
                return carry

            lax.fori_loop(0, TROWS // 16, zrow_body, 0)

            def zdeg_body(r, carry):
                pltpu.sync_copy(zdeg_v, deg_sh.at[pl.ds(row0 + r * 64, 64)])
                return carry

            lax.fori_loop(0, TROWS // 64, zdeg_body, 0)
            plsc.subcore_barrier()

            def _edges_start(i, gidx_v, sidx_v, w_v, sem_e):
                e0 = pl.multiple_of((s + i * 16) * _EBLK, _EBLK)
                pltpu.async_copy(gat_r.at[pl.ds(e0, _EBLK)], gidx_v, sem_e)
                pltpu.async_copy(red_r.at[pl.ds(e0, _EBLK)], sidx_v, sem_e)
                pltpu.async_copy(w_r.at[pl.ds(e0, _EBLK)], w_v, sem_e)

            def _edges_wait(i, gidx_v, sidx_v, w_v, sem_e):
                e0 = pl.multiple_of((s + i * 16) * _EBLK, _EBLK)
                pltpu.make_async_copy(gat_r.at[pl.ds(e0, _EBLK)], gidx_v, sem_e).wait()
                pltpu.make_async_copy(red_r.at[pl.ds(e0, _EBLK)], sidx_v, sem_e).wait()
                pltpu.make_async_copy(w_r.at[pl.ds(e0, _EBLK)], w_v, sem_e).wait()

            def grp_body(g, carry):
                i0 = 2 * g
                i1 = 2 * g + 1

                @pl.when(g > 0)
                def _():
                    pltpu.make_async_copy(rows_a, acc_sh.at[sidx_a], sem_sa).wait()
                    pltpu.make_async_copy(w_a, deg_sh.at[sidx_a], sem_sa).wait()

                _edges_start(i0, gidx_a, sidx_a, w_a, sem_ea)

                @pl.when(i1 < nb)
                def _():
                    @pl.when(g > 0)
                    def _():
                        pltpu.make_async_copy(rows_b, acc_sh.at[sidx_b], sem_sb).wait()
                        pltpu.make_async_copy(w_b, deg_sh.at[sidx_b], sem_sb).wait()

                    _edges_start(i1, gidx_b, sidx_b, w_b, sem_eb)

                _edges_wait(i0, gidx_a, sidx_a, w_a, sem_ea)
                _prep(gidx_a, sidx_a, w_a)
                pltpu.async_copy(xin_r.at[gidx_a], rows_a, sem_ga)

                @pl.when(i1 < nb)
                def _():
                    _edges_wait(i1, gidx_b, sidx_b, w_b, sem_eb)
                    _prep(gidx_b, sidx_b, w_b)
                    pltpu.async_copy(xin_r.at[gidx_b], rows_b, sem_gb)

                pltpu.make_async_copy(xin_r.at[gidx_a], rows_a, sem_ga).wait()
                _scale(rows_a, w_a)
                pltpu.async_copy(rows_a, acc_sh.at[sidx_a], sem_sa, add=True)
                pltpu.async_copy(w_a, deg_sh.at[sidx_a], sem_sa, add=True)

                @pl.when(i1 < nb)
                def _():
                    pltpu.make_async_copy(xin_r.at[gidx_b], rows_b, sem_gb).wait()
                    _scale(rows_b, w_b)
                    pltpu.async_copy(rows_b, acc_sh.at[sidx_b], sem_sb, add=True)
                    pltpu.async_copy(w_b, deg_sh.at[sidx_b], sem_sb, add=True)
                return carry

            lax.fori_loop(0, ng, grp_body, 0)
            pltpu.make_async_copy(rows_a, acc_sh.at[sidx_a], sem_sa).wait()
            pltpu.make_async_copy(w_a, deg_sh.at[sidx_a], sem_sa).wait()
            pltpu.make_async_copy(rows_b, acc_sh.at[sidx_b], sem_sb).wait()
            pltpu.make_async_copy(w_b, deg_sh.at[sidx_b], sem_sb).wait()
            plsc.subcore_barrier()
            out0 = pl.multiple_of(c * HP + row0, 8)
            pltpu.sync_copy(acc_sh.at[pl.ds(row0, TROWS)], acc_o.at[pl.ds(out0, TROWS)])
            pltpu.sync_copy(deg_sh.at[pl.ds(row0, TROWS)], deg_o.at[pl.ds(out0, TROWS)])

        _phase(src_r, dst_r, accf_o, degi_o)
        _phase(dst_r, src_r, accb_o, dego_o)

    mesh = plsc.VectorSubcoreMesh(core_axis_name="c", subcore_axis_name="s")
    return pl.kernel(
        body,
        mesh=mesh,
        compiler_params=pltpu.CompilerParams(use_tc_tiling_on_sc=False),
        out_type=[
            jax.ShapeDtypeStruct((2 * HP, _HID), jnp.float32),
            jax.ShapeDtypeStruct((2 * HP,), jnp.float32),
            jax.ShapeDtypeStruct((2 * HP, _HID), jnp.float32),
            jax.ShapeDtypeStruct((2 * HP,), jnp.float32),
        ],
        scratch_types=[
            pltpu.VMEM((_EBLK,), jnp.int32),
            pltpu.VMEM((_EBLK,), jnp.int32),
            pltpu.VMEM((_EBLK,), jnp.float32),
            pltpu.VMEM((_EBLK, _HID), jnp.float32),
            pltpu.VMEM((_EBLK,), jnp.int32),
            pltpu.VMEM((_EBLK,), jnp.int32),
            pltpu.VMEM((_EBLK,), jnp.float32),
            pltpu.VMEM((_EBLK, _HID), jnp.float32),
            pltpu.VMEM((16, _HID), jnp.float32),
            pltpu.VMEM((64,), jnp.float32),
            pltpu.VMEM_SHARED((HP, _HID), jnp.float32),
            pltpu.VMEM_SHARED((HP,), jnp.float32),
            pltpu.SemaphoreType.DMA,
            pltpu.SemaphoreType.DMA,
            pltpu.SemaphoreType.DMA,
            pltpu.SemaphoreType.DMA,
            pltpu.SemaphoreType.DMA,
            pltpu.SemaphoreType.DMA,
        ],
    )(xin, src, dst, w)


def _post_body(af_ref, ab_ref, di_ref, do_ref, h_ref,
               wf1_ref, wf2_ref, bf_ref, wmu_ref, wsg_ref, bsg_ref,
               wpi_ref, bpi_ref, gmm_ref, o1_ref):
    di = di_ref[...]
    do = do_ref[...]
    of = af_ref[...] * (1.0 / jnp.where(di == 0.0, 1.0, di))
    ob = ab_ref[...] * (1.0 / jnp.where(do == 0.0, 1.0, do))
    out = (jnp.dot(of, wf1_ref[...], preferred_element_type=jnp.float32)
           + jnp.dot(ob, wf2_ref[...], preferred_element_type=jnp.float32)
           + bf_ref[...])
    o1 = jnp.concatenate([out, h_ref[...]], axis=-1)
    o1_ref[...] = o1
    mu = jnp.dot(o1, wmu_ref[...], preferred_element_type=jnp.float32)
    zs = jnp.dot(o1, wsg_ref[...], preferred_element_type=jnp.float32) + bsg_ref[...]
    sigma = jnp.maximum(zs, 0.0) + jnp.log1p(jnp.exp(-jnp.abs(zs)))
    zp = jnp.dot(o1, wpi_ref[...], preferred_element_type=jnp.float32) + bpi_ref[...]
    zp = zp - jnp.max(zp, axis=-1, keepdims=True)
    ez = jnp.exp(zp)
    pi = ez / jnp.sum(ez, axis=-1, keepdims=True)
    gmm_ref[...] = jnp.concatenate([mu, sigma, pi], axis=-1)


def _post_tc(af, ab, di, do, h2, wf1, wf2, bf, wmu, wsg, bsg, wpi, bpi, n):
    HP = af.shape[0] // 2
    H = n // 2
    realb = H // _BPOST
    padb = (HP - H) // _BPOST
    pad_map = lambda i: (i + padb * (i >= realb).astype(jnp.int32), 0)
    full = lambda r, c: pl.BlockSpec((r, c), lambda i: (0, 0))
    blk = lambda c: pl.BlockSpec((_BPOST, c), lambda i: (i, 0))
    pblk = lambda c: pl.BlockSpec((_BPOST, c), pad_map)
    return pl.pallas_call(
        _post_body,
        grid=(n // _BPOST,),
        in_specs=[
            pblk(_HID), pblk(_HID), pblk(1), pblk(1), blk(_HID),
            full(_HID, _HID), full(_HID, _HID), full(1, _HID),
            full(2 * _HID, _HID), full(2 * _HID, _HID), full(1, _HID),
            full(2 * _HID, _HID), full(1, _HID),
        ],
        out_specs=[blk(3 * _HID), blk(2 * _HID)],
        out_shape=[
            jax.ShapeDtypeStruct((n, 3 * _HID), jnp.float32),
            jax.ShapeDtypeStruct((n, 2 * _HID), jnp.float32),
        ],
    )(af, ab, di, do, h2, wf1, wf2, bf, wmu, wsg, bsg, wpi, bpi)


def kernel(x, x_hat_1, h, edge_index, edge_weight,
           W_in, b_in, W_filt, b_filt, W_mu, W_sigma, b_sigma, W_pi, b_pi):
    n = x.shape[0]
    h2 = h.reshape(n, _HID)
    xin = _pre_tc(x, x_hat_1, h2, W_in, b_in)
    src = edge_index[0]
    dst = edge_index[1]
    accf, degi, accb, dego = _edge_kernel(xin, src, dst, edge_weight, n)
    gmm, out1 = _post_tc(
        accf, accb, degi.reshape(-1, 1), dego.reshape(-1, 1), h2,
        W_filt[:_HID], W_filt[_HID:], b_filt.reshape(1, _HID),
        W_mu, W_sigma, b_sigma.reshape(1, _HID), W_pi, b_pi.reshape(1, _HID), n)
    return gmm, out1.reshape(n, 1, 2 * _HID), h
